# baseline (speedup 1.0000x reference)
import jax
import jax.numpy as jnp
from jax.experimental import pallas as pl

B = 4096
TEXT_IN = 768
VIS_IN = 512
E = 64
R_CB = 16
R_FU = 32
ALPHA = 0.1
K = 8192
L = 3

TILE_B = 256
NB = B // TILE_B

_F32 = jnp.float32


def _dot256(x, w):
    k = x.shape[1]
    if k <= 256:
        return jnp.dot(x, w, preferred_element_type=_F32)
    acc = jnp.dot(x[:, :256], w[:256, :], preferred_element_type=_F32)
    for s in range(256, k, 256):
        acc = acc + jnp.dot(x[:, s:s + 256], w[s:s + 256, :],
                            preferred_element_type=_F32)
    return acc


def _mlp3(x, Ws, bs, one=None):
    pin = (lambda v: v * one) if one is not None else (lambda v: v)
    h = pin(jnp.maximum(_dot256(x, Ws[0]) + bs[0], 0.0))
    h = pin(jnp.maximum(_dot256(h, Ws[1]) + bs[1], 0.0))
    return pin(_dot256(h, Ws[2]) + bs[2])


def _rq_levels(z, aTs, a1s, a2s, a3s, bTs, bs, it_refs, sse_refs):
    res = z
    xhat = jnp.zeros_like(z)
    for l in range(L):
        cbT = jnp.dot(bTs[l], aTs[l], preferred_element_type=_F32)
        sumcb = jnp.sum(cbT * cbT, axis=0, keepdims=True)
        t = jnp.dot(res, cbT, preferred_element_type=_F32)
        sumres = jnp.sum(res * res, axis=1, keepdims=True)
        d = (sumres - 2.0 * t) + sumcb
        idx = jnp.argmin(d, axis=1)
        iota = jax.lax.broadcasted_iota(jnp.int32, (TILE_B, K), 1)
        oh = (iota == idx[:, None]).astype(jnp.bfloat16)
        g = (jnp.dot(oh, a1s[l], preferred_element_type=_F32)
             + jnp.dot(oh, a2s[l], preferred_element_type=_F32)) \
            + jnp.dot(oh, a3s[l], preferred_element_type=_F32)
        q = jnp.dot(g, bs[l], preferred_element_type=_F32)
        dq = q - res
        sse = jnp.sum(jnp.sum(dq * dq, axis=1, keepdims=True), axis=0,
                      keepdims=True)
        it_refs[l][0, 0, :] = idx
        sse_refs[l][0, :, :] = jnp.broadcast_to(sse, (1, 128))
        res = res - q
        xhat = xhat + q
    return xhat


def _fused_body(*refs):
    (xt_ref, xv_ref, one_ref,
     tW0, tW1, tW2, tb0, tb1, tb2,
     vW0, vW1, vW2, vb0, vb1, vb2,
     tfA, tfB, vfA, vfB,
     aT0, aT1, aT2,
     a10, a11, a12, a20, a21, a22, a30, a31, a32,
     tbT0, tbT1, tbT2, tbb0, tbb1, tbb2,
     vbT0, vbT1, vbT2, vbb0, vbb1, vbb2,
     tdW0, tdW1, tdW2, tdb0, tdb1, tdb2,
     vdW0, vdW1, vdW2, vdb0, vdb1, vdb2,
     out_t, out_v,
     it0, it1, it2, iv0, iv1, iv2,
     st0, st1, st2, sv0, sv1, sv2) = refs

    xt = xt_ref[...]
    xv = xv_ref[...]
    one = one_ref[...]

    z_text = _mlp3(xt, (tW0[...], tW1[...], tW2[...]),
                   (tb0[...], tb1[...], tb2[...]), one)
    z_vis = _mlp3(xv, (vW0[...], vW1[...], vW2[...]),
                  (vb0[...], vb1[...], vb2[...]), one)

    delta_text = jnp.dot(jnp.dot(z_vis, vfB[...], preferred_element_type=_F32),
                         tfA[...], preferred_element_type=_F32)
    z_tf = z_text + ALPHA * delta_text
    delta_vis = jnp.dot(jnp.dot(z_text, tfB[...], preferred_element_type=_F32),
                        vfA[...], preferred_element_type=_F32)
    z_vf = z_vis + ALPHA * delta_vis

    aTs = (aT0[...], aT1[...], aT2[...])
    a1s = (a10[...], a11[...], a12[...])
    a2s = (a20[...], a21[...], a22[...])
    a3s = (a30[...], a31[...], a32[...])

    xhat_t = _rq_levels(z_tf, aTs, a1s, a2s, a3s,
                        (tbT0[...], tbT1[...], tbT2[...]),
                        (tbb0[...], tbb1[...], tbb2[...]),
                        (it0, it1, it2), (st0, st1, st2))
    xhat_v = _rq_levels(z_vf, aTs, a1s, a2s, a3s,
                        (vbT0[...], vbT1[...], vbT2[...]),
                        (vbb0[...], vbb1[...], vbb2[...]),
                        (iv0, iv1, iv2), (sv0, sv1, sv2))

    out_t[...] = _mlp3(xhat_t, (tdW0[...], tdW1[...], tdW2[...]),
                       (tdb0[...], tdb1[...], tdb2[...]))
    out_v[...] = _mlp3(xhat_v, (vdW0[...], vdW1[...], vdW2[...]),
                       (vdb0[...], vdb1[...], vdb2[...]))


def _split3(a):
    a1 = a.astype(jnp.bfloat16)
    r1 = a - a1.astype(_F32)
    a2 = r1.astype(jnp.bfloat16)
    a3 = (r1 - a2.astype(_F32)).astype(jnp.bfloat16)
    return a1, a2, a3


def kernel(x_text, x_vis, params):
    p = params
    row2 = lambda b: b.reshape(1, -1)

    a_list = p["cb_a"]
    aTs = [a.T for a in a_list]
    splits = [_split3(a) for a in a_list]
    a1s = [s[0] for s in splits]
    a2s = [s[1] for s in splits]
    a3s = [s[2] for s in splits]

    inputs = [x_text, x_vis, jnp.ones((1, 1), _F32)]
    inputs += list(p["text_enc_W"]) + [row2(b) for b in p["text_enc_b"]]
    inputs += list(p["vis_enc_W"]) + [row2(b) for b in p["vis_enc_b"]]
    inputs += [p["text_fA"], p["text_fB"], p["vis_fA"], p["vis_fB"]]
    inputs += aTs + a1s + a2s + a3s
    inputs += [b.T for b in p["text_cb_b"]] + list(p["text_cb_b"])
    inputs += [b.T for b in p["vis_cb_b"]] + list(p["vis_cb_b"])
    inputs += list(p["text_dec_W"]) + [row2(b) for b in p["text_dec_b"]]
    inputs += list(p["vis_dec_W"]) + [row2(b) for b in p["vis_dec_b"]]

    def tile_spec(shape):
        return pl.BlockSpec((TILE_B,) + shape[1:], lambda i: (i,) + (0,) * (len(shape) - 1))

    def whole_spec(shape):
        nd = len(shape)
        return pl.BlockSpec(shape, lambda i, _n=nd: (0,) * _n)

    in_specs = [tile_spec(x_text.shape), tile_spec(x_vis.shape)]
    in_specs += [whole_spec(a.shape) for a in inputs[2:]]

    out_shapes = [
        jax.ShapeDtypeStruct((B, TEXT_IN), _F32),
        jax.ShapeDtypeStruct((B, VIS_IN), _F32),
    ]
    out_specs = [tile_spec((B, TEXT_IN)), tile_spec((B, VIS_IN))]
    for _ in range(6):
        out_shapes.append(jax.ShapeDtypeStruct((NB, 1, TILE_B), jnp.int32))
        out_specs.append(pl.BlockSpec((1, 1, TILE_B), lambda i: (i, 0, 0)))
    for _ in range(6):
        out_shapes.append(jax.ShapeDtypeStruct((NB, 1, 128), _F32))
        out_specs.append(pl.BlockSpec((1, 1, 128), lambda i: (i, 0, 0)))

    outs = pl.pallas_call(
        _fused_body,
        grid=(NB,),
        in_specs=in_specs,
        out_specs=out_specs,
        out_shape=out_shapes,
    )(*inputs)

    out_text, out_vis = outs[0], outs[1]
    it = jnp.stack([outs[2 + l].reshape(B) for l in range(L)], axis=-1)
    iv = jnp.stack([outs[5 + l].reshape(B) for l in range(L)], axis=-1)
    sse_t = [jnp.sum(outs[8 + l][:, 0, 0]) for l in range(L)]
    sse_v = [jnp.sum(outs[11 + l][:, 0, 0]) for l in range(L)]
    denom = float(B * E)
    lt = jnp.mean(jnp.stack([1.25 * s / denom for s in sse_t]))
    lv = jnp.mean(jnp.stack([1.25 * s / denom for s in sse_v]))
    return (out_text, out_vis, lt, lv, it, iv)


# speedup vs baseline: 1.9707x; 1.9707x over previous
import jax
import jax.numpy as jnp
from jax.experimental import pallas as pl

B = 4096
TEXT_IN = 768
VIS_IN = 512
E = 64
R_CB = 16
R_FU = 32
ALPHA = 0.1
K = 8192
L = 3

TILE_B = 256
NB = B // TILE_B

_F32 = jnp.float32


def _dot256(x, w):
    k = x.shape[1]
    if k <= 256:
        return jnp.dot(x, w, preferred_element_type=_F32)
    acc = jnp.dot(x[:, :256], w[:256, :], preferred_element_type=_F32)
    for s in range(256, k, 256):
        acc = acc + jnp.dot(x[:, s:s + 256], w[s:s + 256, :],
                            preferred_element_type=_F32)
    return acc


def _mlp3(x, Ws, bs, one=None):
    pin = (lambda v: v * one) if one is not None else (lambda v: v)
    h = pin(jnp.maximum(_dot256(x, Ws[0]) + bs[0], 0.0))
    h = pin(jnp.maximum(_dot256(h, Ws[1]) + bs[1], 0.0))
    return pin(_dot256(h, Ws[2]) + bs[2])


def _rq_levels(z, aTs, acats, bTs, bs, it_refs, sse_refs):
    res = z
    xhat = jnp.zeros_like(z)
    for l in range(L):
        cbT = jnp.dot(bTs[l], aTs[l], preferred_element_type=_F32)
        sumcb = jnp.sum(cbT * cbT, axis=0, keepdims=True)
        t = jnp.dot(res, cbT, preferred_element_type=_F32)
        sumres = jnp.sum(res * res, axis=1, keepdims=True)
        d = (sumres - 2.0 * t) + sumcb
        idx = jnp.argmin(d, axis=1)
        iota = jax.lax.broadcasted_iota(jnp.int32, (TILE_B, K), 1)
        oh = (iota == idx[:, None]).astype(jnp.bfloat16)
        g3 = jnp.dot(oh, acats[l], preferred_element_type=_F32)
        g = (g3[:, :R_CB] + g3[:, R_CB:2 * R_CB]) + g3[:, 2 * R_CB:]
        q = jnp.dot(g, bs[l], preferred_element_type=_F32)
        dq = q - res
        sse = jnp.sum(jnp.sum(dq * dq, axis=1, keepdims=True), axis=0,
                      keepdims=True)
        it_refs[l][0, 0, :] = idx
        sse_refs[l][0, :, :] = jnp.broadcast_to(sse, (1, 128))
        res = res - q
        xhat = xhat + q
    return xhat


def _fused_body(*refs):
    (xt_ref, xv_ref, one_ref,
     tW0, tW1, tW2, tb0, tb1, tb2,
     vW0, vW1, vW2, vb0, vb1, vb2,
     tfA, tfB, vfA, vfB,
     aT0, aT1, aT2,
     ac0, ac1, ac2,
     tbT0, tbT1, tbT2, tbb0, tbb1, tbb2,
     vbT0, vbT1, vbT2, vbb0, vbb1, vbb2,
     tdW0, tdW1, tdW2, tdb0, tdb1, tdb2,
     vdW0, vdW1, vdW2, vdb0, vdb1, vdb2,
     out_t, out_v,
     it0, it1, it2, iv0, iv1, iv2,
     st0, st1, st2, sv0, sv1, sv2) = refs

    xt = xt_ref[...]
    xv = xv_ref[...]
    one = one_ref[...]

    z_text = _mlp3(xt, (tW0[...], tW1[...], tW2[...]),
                   (tb0[...], tb1[...], tb2[...]), one)
    z_vis = _mlp3(xv, (vW0[...], vW1[...], vW2[...]),
                  (vb0[...], vb1[...], vb2[...]), one)

    delta_text = jnp.dot(jnp.dot(z_vis, vfB[...], preferred_element_type=_F32),
                         tfA[...], preferred_element_type=_F32)
    z_tf = z_text + ALPHA * delta_text
    delta_vis = jnp.dot(jnp.dot(z_text, tfB[...], preferred_element_type=_F32),
                        vfA[...], preferred_element_type=_F32)
    z_vf = z_vis + ALPHA * delta_vis

    aTs = (aT0[...], aT1[...], aT2[...])
    acats = (ac0[...], ac1[...], ac2[...])

    xhat_t = _rq_levels(z_tf, aTs, acats,
                        (tbT0[...], tbT1[...], tbT2[...]),
                        (tbb0[...], tbb1[...], tbb2[...]),
                        (it0, it1, it2), (st0, st1, st2))
    xhat_v = _rq_levels(z_vf, aTs, acats,
                        (vbT0[...], vbT1[...], vbT2[...]),
                        (vbb0[...], vbb1[...], vbb2[...]),
                        (iv0, iv1, iv2), (sv0, sv1, sv2))

    out_t[...] = _mlp3(xhat_t, (tdW0[...], tdW1[...], tdW2[...]),
                       (tdb0[...], tdb1[...], tdb2[...]))
    out_v[...] = _mlp3(xhat_v, (vdW0[...], vdW1[...], vdW2[...]),
                       (vdb0[...], vdb1[...], vdb2[...]))


def _split3(a):
    a1 = a.astype(jnp.bfloat16)
    r1 = a - a1.astype(_F32)
    a2 = r1.astype(jnp.bfloat16)
    a3 = (r1 - a2.astype(_F32)).astype(jnp.bfloat16)
    return a1, a2, a3


def kernel(x_text, x_vis, params):
    p = params
    row2 = lambda b: b.reshape(1, -1)

    a_list = p["cb_a"]
    aTs = [a.T for a in a_list]
    acats = [jnp.concatenate(_split3(a), axis=1) for a in a_list]

    inputs = [x_text, x_vis, jnp.ones((1, 1), _F32)]
    inputs += list(p["text_enc_W"]) + [row2(b) for b in p["text_enc_b"]]
    inputs += list(p["vis_enc_W"]) + [row2(b) for b in p["vis_enc_b"]]
    inputs += [p["text_fA"], p["text_fB"], p["vis_fA"], p["vis_fB"]]
    inputs += aTs + acats
    inputs += [b.T for b in p["text_cb_b"]] + list(p["text_cb_b"])
    inputs += [b.T for b in p["vis_cb_b"]] + list(p["vis_cb_b"])
    inputs += list(p["text_dec_W"]) + [row2(b) for b in p["text_dec_b"]]
    inputs += list(p["vis_dec_W"]) + [row2(b) for b in p["vis_dec_b"]]

    def tile_spec(shape):
        return pl.BlockSpec((TILE_B,) + shape[1:], lambda i: (i,) + (0,) * (len(shape) - 1))

    def whole_spec(shape):
        nd = len(shape)
        return pl.BlockSpec(shape, lambda i, _n=nd: (0,) * _n)

    in_specs = [tile_spec(x_text.shape), tile_spec(x_vis.shape)]
    in_specs += [whole_spec(a.shape) for a in inputs[2:]]

    out_shapes = [
        jax.ShapeDtypeStruct((B, TEXT_IN), _F32),
        jax.ShapeDtypeStruct((B, VIS_IN), _F32),
    ]
    out_specs = [tile_spec((B, TEXT_IN)), tile_spec((B, VIS_IN))]
    for _ in range(6):
        out_shapes.append(jax.ShapeDtypeStruct((NB, 1, TILE_B), jnp.int32))
        out_specs.append(pl.BlockSpec((1, 1, TILE_B), lambda i: (i, 0, 0)))
    for _ in range(6):
        out_shapes.append(jax.ShapeDtypeStruct((NB, 1, 128), _F32))
        out_specs.append(pl.BlockSpec((1, 1, 128), lambda i: (i, 0, 0)))

    outs = pl.pallas_call(
        _fused_body,
        grid=(NB,),
        in_specs=in_specs,
        out_specs=out_specs,
        out_shape=out_shapes,
    )(*inputs)

    out_text, out_vis = outs[0], outs[1]
    it = jnp.stack([outs[2 + l].reshape(B) for l in range(L)], axis=-1)
    iv = jnp.stack([outs[5 + l].reshape(B) for l in range(L)], axis=-1)
    sse_t = [jnp.sum(outs[8 + l][:, 0, 0]) for l in range(L)]
    sse_v = [jnp.sum(outs[11 + l][:, 0, 0]) for l in range(L)]
    denom = float(B * E)
    lt = jnp.mean(jnp.stack([1.25 * s / denom for s in sse_t]))
    lv = jnp.mean(jnp.stack([1.25 * s / denom for s in sse_v]))
    return (out_text, out_vis, lt, lv, it, iv)
